# baseline (speedup 1.0000x reference)
import functools

import jax
import jax.numpy as jnp
from jax import lax
from jax.experimental import pallas as pl
from jax.experimental.pallas import tpu as pltpu
from jax.experimental.pallas import tpu_sc as plsc

HIDDEN = 1024
NC = 2
NS = 16
NW = NC * NS
R = 32
NBUF = 3


@functools.lru_cache(maxsize=None)
def _make_sc_gather(B):
    assert B % NW == 0
    b_per_w = B // NW
    assert b_per_w % R == 0
    C = b_per_w // R
    assert C >= NBUF
    groups = (C + NBUF - 1) // NBUF

    mesh = plsc.VectorSubcoreMesh(core_axis_name="c", subcore_axis_name="s")

    @functools.partial(
        pl.kernel,
        out_type=jax.ShapeDtypeStruct((B, HIDDEN), jnp.float32),
        mesh=mesh,
        scratch_types=[
            pltpu.VMEM((b_per_w,), jnp.int32),
            pltpu.VMEM((R, HIDDEN), jnp.float32),
            pltpu.VMEM((R, HIDDEN), jnp.float32),
            pltpu.VMEM((R, HIDDEN), jnp.float32),
            pltpu.SemaphoreType.DMA,
            pltpu.SemaphoreType.DMA,
            pltpu.SemaphoreType.DMA,
            pltpu.SemaphoreType.DMA,
            pltpu.SemaphoreType.DMA,
            pltpu.SemaphoreType.DMA,
        ],
    )
    def gather_kernel(idx_hbm, table_hbm, out_hbm, idx_v,
                      buf0, buf1, buf2, g0, g1, g2, w0, w1, w2):
        wid = lax.axis_index("s") * NC + lax.axis_index("c")
        base = wid * b_per_w
        pltpu.sync_copy(idx_hbm.at[pl.ds(base, b_per_w)], idx_v)

        bufs = (buf0, buf1, buf2)
        gsems = (g0, g1, g2)
        wsems = (w0, w1, w2)

        def start_g(c, b):
            pltpu.make_async_copy(
                table_hbm.at[idx_v.at[pl.ds(c * R, R)]], bufs[b], gsems[b]
            ).start()

        def wait_g(b):
            pltpu.make_async_copy(
                table_hbm.at[idx_v.at[pl.ds(0, R)]], bufs[b], gsems[b]
            ).wait()

        def start_w(c, b):
            pltpu.make_async_copy(
                bufs[b], out_hbm.at[pl.ds(base + c * R, R)], wsems[b]
            ).start()

        def wait_w(b):
            pltpu.make_async_copy(
                bufs[b], out_hbm.at[pl.ds(base, R)], wsems[b]
            ).wait()

        for j in range(NBUF - 1):
            start_g(j, j)

        def body(i, carry):
            for s in range(NBUF):
                c = i * NBUF + s

                @pl.when(c < C)
                def _():
                    wait_g(s)
                    start_w(c, s)
                    nxt = c + NBUF - 1
                    bj = (s + NBUF - 1) % NBUF

                    @pl.when(jnp.logical_and(nxt < C, c >= 1))
                    def _():
                        wait_w(bj)
                        start_g(nxt, bj)

                    @pl.when(jnp.logical_and(nxt < C, c < 1))
                    def _():
                        start_g(nxt, bj)

            return carry

        lax.fori_loop(0, groups, body, 0, unroll=False)

        for s in range(NBUF):
            wait_w((C - NBUF + s) % NBUF)

    return gather_kernel


def kernel(position_ids, pos_enc):
    batch, seq = position_ids.shape
    B = batch * seq
    idx = position_ids.reshape(B).astype(jnp.int32)
    out = _make_sc_gather(B)(idx, pos_enc)
    return out.reshape(batch, seq, HIDDEN)
